# baseline (speedup 1.0000x reference)
import functools

import jax
import jax.numpy as jnp
from jax import lax
from jax.experimental import pallas as pl
from jax.experimental.pallas import tpu as pltpu
from jax.experimental.pallas import tpu_sc as plsc

NUM_EXPERTS = 64
TOP_K = 8
BLOCK_ROWS = 1024

NEG_INF = float("-inf")


def _matmul_kernel(hs_ref, wt_ref, logits_ref, lt_ref):
    hs = hs_ref[:]
    logits = jnp.dot(hs, wt_ref[:], preferred_element_type=jnp.float32)
    logits_ref[:] = logits
    lt_ref[:] = logits.T


@jax.jit
def _matmul(hs, wt):
    n = hs.shape[0]
    grid = (n // BLOCK_ROWS,)
    return pl.pallas_call(
        _matmul_kernel,
        grid=grid,
        in_specs=[
            pl.BlockSpec((BLOCK_ROWS, hs.shape[1]), lambda i: (i, 0)),
            pl.BlockSpec((hs.shape[1], NUM_EXPERTS), lambda i: (0, 0)),
        ],
        out_specs=[
            pl.BlockSpec((BLOCK_ROWS, NUM_EXPERTS), lambda i: (i, 0)),
            pl.BlockSpec((NUM_EXPERTS, BLOCK_ROWS), lambda i: (0, i)),
        ],
        out_shape=[
            jax.ShapeDtypeStruct((n, NUM_EXPERTS), jnp.float32),
            jax.ShapeDtypeStruct((NUM_EXPERTS, n), jnp.float32),
        ],
    )(hs, wt)


def _tournament(vals, idxs):
    while len(vals) > 1:
        nv, ni = [], []
        for j in range(0, len(vals), 2):
            a, b = vals[j], vals[j + 1]
            ge = a >= b
            nv.append(jnp.where(ge, a, b))
            ni.append(jnp.where(ge, idxs[j], idxs[j + 1]))
        vals, idxs = nv, ni
    return vals[0], idxs[0]


def _make_sc_topk(n):
    nw = 32
    rpw = n // nw
    cb = 256
    mesh = plsc.VectorSubcoreMesh(core_axis_name="c", subcore_axis_name="s")

    @functools.partial(
        pl.kernel,
        mesh=mesh,
        out_type=[
            jax.ShapeDtypeStruct((TOP_K, n), jnp.int32),
            jax.ShapeDtypeStruct((TOP_K, n), jnp.float32),
        ],
        scratch_types=[
            pltpu.VMEM((NUM_EXPERTS, cb), jnp.float32),
            pltpu.VMEM((TOP_K, cb), jnp.int32),
            pltpu.VMEM((TOP_K, cb), jnp.float32),
        ],
    )
    def sc_topk(lt_hbm, idx_hbm, w_hbm, buf, oi, ow):
        wid = lax.axis_index("s") * 2 + lax.axis_index("c")
        base = wid * rpw

        def chunk_body(ci, carry):
            cbase = base + ci * cb
            pltpu.sync_copy(lt_hbm.at[:, pl.ds(cbase, cb)], buf)

            def group_body(g, carry2):
                off = g * 16
                work = [buf[e, pl.ds(off, 16)] for e in range(NUM_EXPERTS)]
                eidx = [jnp.full((16,), e, jnp.int32) for e in range(NUM_EXPERTS)]
                topv = []
                for k in range(TOP_K):
                    mv, mi = _tournament(list(work), eidx)
                    oi[k, pl.ds(off, 16)] = mi
                    topv.append(mv)
                    if k < TOP_K - 1:
                        work = [
                            jnp.where(mi == e, NEG_INF, work[e])
                            for e in range(NUM_EXPERTS)
                        ]
                es = [jnp.exp(v - topv[0]) for v in topv]
                tot = es[0]
                for e in es[1:]:
                    tot = tot + e
                for k in range(TOP_K):
                    ow[k, pl.ds(off, 16)] = es[k] / tot
                return carry2

            lax.fori_loop(0, cb // 16, group_body, 0)
            pltpu.sync_copy(oi, idx_hbm.at[:, pl.ds(cbase, cb)])
            pltpu.sync_copy(ow, w_hbm.at[:, pl.ds(cbase, cb)])
            return carry

        lax.fori_loop(0, rpw // cb, chunk_body, 0)

    return sc_topk


@jax.jit
def _router(hs, wt):
    logits, lt = _matmul(hs, wt)
    topi, topw = _make_sc_topk(hs.shape[0])(lt)
    return logits, topi, topw


def kernel(hidden_states, weight):
    hs = hidden_states.reshape(-1, hidden_states.shape[-1]).astype(jnp.float32)
    wt = weight.astype(jnp.float32)
    logits, topi, topw = _router(hs, wt.T)
    return (topi.T, topw.T, logits)


# speedup vs baseline: 1.4136x; 1.4136x over previous
import jax
import jax.numpy as jnp
from jax.experimental import pallas as pl

NUM_EXPERTS = 64
TOP_K = 8
BLOCK_ROWS = 1024
HALF = BLOCK_ROWS // 2


def _router_kernel(hs0_ref, hs1_ref, wt_ref, logits_ref, idx_ref, w_ref):
    wt = wt_ref[:]
    l0 = jnp.dot(hs0_ref[:], wt, preferred_element_type=jnp.float32)
    l1 = jnp.dot(hs1_ref[:], wt, preferred_element_type=jnp.float32)
    logits_ref[:HALF] = l0
    logits_ref[HALF:] = l1

    work = jnp.concatenate([l0.T, l1.T], axis=1)

    rows = work.shape[1]
    iota = jax.lax.broadcasted_iota(jnp.int32, (NUM_EXPERTS, rows), 0)
    vals = []
    idxs = []
    for _ in range(TOP_K):
        m = jnp.max(work, axis=0, keepdims=True)
        cand = jnp.where(work == m, iota, NUM_EXPERTS)
        idx = jnp.min(cand, axis=0, keepdims=True)
        vals.append(m)
        idxs.append(idx)
        work = jnp.where(cand == idx, -jnp.inf, work)

    topv = jnp.concatenate(vals, axis=0)
    topi = jnp.concatenate(idxs, axis=0)
    e = jnp.exp(topv - topv[0:1, :])
    w_ref[:] = e / jnp.sum(e, axis=0, keepdims=True)
    idx_ref[:] = topi


@jax.jit
def _router(hs, wt):
    n = hs.shape[0]
    grid = (n // BLOCK_ROWS,)
    return pl.pallas_call(
        _router_kernel,
        grid=grid,
        in_specs=[
            pl.BlockSpec((HALF, hs.shape[1]), lambda i: (2 * i, 0)),
            pl.BlockSpec((HALF, hs.shape[1]), lambda i: (2 * i + 1, 0)),
            pl.BlockSpec((hs.shape[1], NUM_EXPERTS), lambda i: (0, 0)),
        ],
        out_specs=[
            pl.BlockSpec((BLOCK_ROWS, NUM_EXPERTS), lambda i: (i, 0)),
            pl.BlockSpec((TOP_K, BLOCK_ROWS), lambda i: (0, i)),
            pl.BlockSpec((TOP_K, BLOCK_ROWS), lambda i: (0, i)),
        ],
        out_shape=[
            jax.ShapeDtypeStruct((n, NUM_EXPERTS), jnp.float32),
            jax.ShapeDtypeStruct((TOP_K, n), jnp.int32),
            jax.ShapeDtypeStruct((TOP_K, n), jnp.float32),
        ],
    )(hs, hs, wt)


def kernel(hidden_states, weight):
    hs = hidden_states.reshape(-1, hidden_states.shape[-1]).astype(jnp.float32)
    wt = weight.astype(jnp.float32)
    logits, topi, topw = _router(hs, wt.T)
    return (topi.T, topw.T, logits)


# speedup vs baseline: 1.4259x; 1.0087x over previous
import jax
import jax.numpy as jnp
from jax.experimental import pallas as pl

NUM_EXPERTS = 64
TOP_K = 8
BLOCK_ROWS = 1024


def _router_kernel(hs_ref, wt_ref, logits_ref, idx_ref, w_ref):
    hs = hs_ref[:]
    logits = jnp.dot(hs, wt_ref[:], preferred_element_type=jnp.float32)
    logits_ref[:] = logits

    work = logits.T

    rows = work.shape[1]
    iota = jax.lax.broadcasted_iota(jnp.int32, (NUM_EXPERTS, rows), 0)
    vals = []
    idxs = []
    for _ in range(TOP_K):
        m = jnp.max(work, axis=0, keepdims=True)
        cand = jnp.where(work == m, iota, NUM_EXPERTS)
        idx = jnp.min(cand, axis=0, keepdims=True)
        vals.append(m)
        idxs.append(idx)
        work = jnp.where(cand == idx, -jnp.inf, work)

    topv = jnp.concatenate(vals, axis=0)
    topi = jnp.concatenate(idxs, axis=0)
    e = jnp.exp(topv - topv[0:1, :])
    w_ref[:] = e / jnp.sum(e, axis=0, keepdims=True)
    idx_ref[:] = topi


@jax.jit
def _router(hs, wt):
    n = hs.shape[0]
    grid = (n // BLOCK_ROWS,)
    return pl.pallas_call(
        _router_kernel,
        grid=grid,
        in_specs=[
            pl.BlockSpec((BLOCK_ROWS, hs.shape[1]), lambda i: (i, 0)),
            pl.BlockSpec((hs.shape[1], NUM_EXPERTS), lambda i: (0, 0)),
        ],
        out_specs=[
            pl.BlockSpec((BLOCK_ROWS, NUM_EXPERTS), lambda i: (i, 0)),
            pl.BlockSpec((TOP_K, BLOCK_ROWS), lambda i: (0, i)),
            pl.BlockSpec((TOP_K, BLOCK_ROWS), lambda i: (0, i)),
        ],
        out_shape=[
            jax.ShapeDtypeStruct((n, NUM_EXPERTS), jnp.float32),
            jax.ShapeDtypeStruct((TOP_K, n), jnp.int32),
            jax.ShapeDtypeStruct((TOP_K, n), jnp.float32),
        ],
    )(hs, wt)


def kernel(hidden_states, weight):
    hs = hidden_states.reshape(-1, hidden_states.shape[-1]).astype(jnp.float32)
    wt = weight.astype(jnp.float32)
    logits, topi, topw = _router(hs, wt.T)
    return (topi.T, topw.T, logits)


# speedup vs baseline: 1.4527x; 1.0188x over previous
import jax
import jax.numpy as jnp
from jax.experimental import pallas as pl

NUM_EXPERTS = 64
TOP_K = 8
BLOCK_ROWS = 1024


def _router_kernel(hs_ref, wt_ref, logits_ref, idx_ref, w_ref):
    hs = hs_ref[:]
    logits = jax.lax.dot_general(
        hs, wt_ref[:], (((1,), (1,)), ((), ())),
        preferred_element_type=jnp.float32)
    logits_ref[:] = logits

    work = logits.T

    rows = work.shape[1]
    iota = jax.lax.broadcasted_iota(jnp.int32, (NUM_EXPERTS, rows), 0)
    vals = []
    idxs = []
    for _ in range(TOP_K):
        m = jnp.max(work, axis=0, keepdims=True)
        cand = jnp.where(work == m, iota, NUM_EXPERTS)
        idx = jnp.min(cand, axis=0, keepdims=True)
        vals.append(m)
        idxs.append(idx)
        work = jnp.where(cand == idx, -jnp.inf, work)

    topv = jnp.concatenate(vals, axis=0)
    topi = jnp.concatenate(idxs, axis=0)
    e = jnp.exp(topv - topv[0:1, :])
    w_ref[:] = e / jnp.sum(e, axis=0, keepdims=True)
    idx_ref[:] = topi


@jax.jit
def _router(hs, wt):
    n = hs.shape[0]
    grid = (n // BLOCK_ROWS,)
    return pl.pallas_call(
        _router_kernel,
        grid=grid,
        in_specs=[
            pl.BlockSpec((BLOCK_ROWS, hs.shape[1]), lambda i: (i, 0)),
            pl.BlockSpec((NUM_EXPERTS, hs.shape[1]), lambda i: (0, 0)),
        ],
        out_specs=[
            pl.BlockSpec((BLOCK_ROWS, NUM_EXPERTS), lambda i: (i, 0)),
            pl.BlockSpec((TOP_K, BLOCK_ROWS), lambda i: (0, i)),
            pl.BlockSpec((TOP_K, BLOCK_ROWS), lambda i: (0, i)),
        ],
        out_shape=[
            jax.ShapeDtypeStruct((n, NUM_EXPERTS), jnp.float32),
            jax.ShapeDtypeStruct((TOP_K, n), jnp.int32),
            jax.ShapeDtypeStruct((TOP_K, n), jnp.float32),
        ],
    )(hs, wt)


def kernel(hidden_states, weight):
    hs = hidden_states.reshape(-1, hidden_states.shape[-1]).astype(jnp.float32)
    wt = weight.astype(jnp.float32)
    logits, topi, topw = _router(hs, wt)
    return (topi.T, topw.T, logits)
